# baseline (speedup 1.0000x reference)
import functools

import jax
import jax.numpy as jnp
from jax import lax
from jax.experimental import pallas as pl
from jax.experimental.pallas import tpu as pltpu
from jax.experimental.pallas import tpu_sc as plsc

LANES = 128
KB = 4


@functools.lru_cache(maxsize=None)
def _make_gather(b: int, s: int, vocab: int, d: int):
    info = plsc.get_sparse_core_info()
    nc, ns = info.num_cores, info.num_subcores
    nw = nc * ns
    assert b % (nw * LANES) == 0 and d % 8 == 0 and s % KB == 0
    nb = s // KB

    @functools.partial(
        pl.kernel,
        mesh=plsc.VectorSubcoreMesh(core_axis_name="c", subcore_axis_name="s"),
        out_type=jax.ShapeDtypeStruct((s, d // 8, b // LANES, 8, LANES), jnp.float32),
        scratch_types=[
            pltpu.VMEM((s, LANES), jnp.int32),
            pltpu.VMEM((2, KB, LANES, d), jnp.float32),
            pltpu.VMEM((2, d // 8, 8, LANES), jnp.float32),
            pltpu.SemaphoreType.DMA,
            pltpu.SemaphoreType.DMA,
        ],
        compiler_params=pltpu.CompilerParams(
            use_tc_tiling_on_sc=False, needs_layout_passes=False
        ),
    )
    def gather_kernel(idx_hbm, table_hbm, out_hbm, idx_v, gbuf, tbuf, gsem, osem):
        wid = lax.axis_index("s") * nc + lax.axis_index("c")
        pltpu.sync_copy(idx_hbm.at[:, pl.ds(wid * LANES, LANES)], idx_v)

        rowv = [
            (jnp.arange(16, dtype=jnp.int32) + 16 * t) for t in range(LANES // 16)
        ]

        def fire(i, p):
            for c in range(KB):
                pltpu.async_copy(
                    table_hbm.at[idx_v.at[i * KB + c]], gbuf.at[p, c], gsem
                )

        def drain(p):
            for c in range(KB):
                pltpu.make_async_copy(
                    table_hbm.at[idx_v.at[0]], gbuf.at[p, c], gsem
                ).wait()

        def wait_out(q):
            pltpu.make_async_copy(out_hbm.at[0, :, wid], tbuf.at[q], osem).wait()

        def emit(i, p, c):
            q = c % 2
            src = gbuf.at[p, c]
            for dd in range(d):
                colv = jnp.full((16,), dd, dtype=jnp.int32)
                vecs = [
                    plsc.load_gather(src, [rowv[t], colv])
                    for t in range(LANES // 16)
                ]
                for t, vec in enumerate(vecs):
                    tbuf[q, dd // 8, dd % 8, pl.ds(t * 16, 16)] = vec
            pltpu.async_copy(tbuf.at[q], out_hbm.at[i * KB + c, :, wid], osem)

        fire(0, 0)

        def body(i, carry):
            p = i % 2
            drain(p)

            @pl.when(i < nb - 1)
            def _():
                fire(i + 1, 1 - p)

            for c in range(KB):
                if c >= 2:
                    wait_out(c % 2)
                else:

                    @pl.when(i > 0)
                    def _():
                        wait_out(c % 2)

                emit(i, p, c)
            return carry

        lax.fori_loop(0, nb, body, 0)
        wait_out(0)
        wait_out(1)

    return gather_kernel


def kernel(gene_ids, table):
    b, s = gene_ids.shape
    vocab, d = table.shape
    idx_t = gene_ids.T.astype(jnp.int32)
    raw = _make_gather(b, s, vocab, d)(idx_t, table)
    emb = raw.transpose(2, 4, 0, 1, 3).reshape(b, s, d)
    return gene_ids, emb


# speedup vs baseline: 1.1127x; 1.1127x over previous
import functools

import jax
import jax.numpy as jnp
from jax import lax
from jax.experimental import pallas as pl
from jax.experimental.pallas import tpu as pltpu
from jax.experimental.pallas import tpu_sc as plsc

LANES = 128
KB = 4


@functools.lru_cache(maxsize=None)
def _make_gather(b: int, s: int, vocab: int, d: int):
    info = plsc.get_sparse_core_info()
    nc, ns = info.num_cores, info.num_subcores
    nw = nc * ns
    assert b % (nw * LANES) == 0 and d % 8 == 0 and s % KB == 0
    nb = s // KB

    @functools.partial(
        pl.kernel,
        mesh=plsc.VectorSubcoreMesh(core_axis_name="c", subcore_axis_name="s"),
        out_type=jax.ShapeDtypeStruct((s, d // 8, b // LANES, 8, LANES), jnp.float32),
        scratch_types=[
            pltpu.VMEM((s, LANES), jnp.int32),
            pltpu.VMEM((2, KB, LANES, d), jnp.float32),
            pltpu.VMEM((2, d // 8, 8, LANES), jnp.float32),
            pltpu.SemaphoreType.DMA,
            pltpu.SemaphoreType.DMA,
        ],
        compiler_params=pltpu.CompilerParams(
            use_tc_tiling_on_sc=False, needs_layout_passes=False
        ),
    )
    def gather_kernel(idx_hbm, table_hbm, out_hbm, idx_v, gbuf, tbuf, gsem, osem):
        wid = lax.axis_index("s") * nc + lax.axis_index("c")
        pltpu.sync_copy(idx_hbm.at[:, pl.ds(wid * LANES, LANES)], idx_v)

        rowv = [
            (jnp.arange(16, dtype=jnp.int32) + 16 * t) for t in range(LANES // 16)
        ]

        def fire(i, p):
            for c in range(KB):
                pltpu.async_copy(
                    table_hbm.at[idx_v.at[i * KB + c]], gbuf.at[p, c], gsem
                )

        def drain(p):
            for c in range(KB):
                pltpu.make_async_copy(
                    table_hbm.at[idx_v.at[0]], gbuf.at[p, c], gsem
                ).wait()

        def wait_out(q):
            pltpu.make_async_copy(out_hbm.at[0, :, wid], tbuf.at[q], osem).wait()

        def emit(i, p, c):
            q = c % 2
            src = gbuf.at[p, c]
            @plsc.parallel_loop(0, d, 1, unroll=8)
            def dloop(dd):
                colv = jnp.full((16,), dd, dtype=jnp.int32)
                vecs = [
                    plsc.load_gather(src, [rowv[t], colv])
                    for t in range(LANES // 16)
                ]
                for t, vec in enumerate(vecs):
                    tbuf[q, dd // 8, dd % 8, pl.ds(t * 16, 16)] = vec
            pltpu.async_copy(tbuf.at[q], out_hbm.at[i * KB + c, :, wid], osem)

        fire(0, 0)

        def body(i, carry):
            p = i % 2
            drain(p)

            @pl.when(i < nb - 1)
            def _():
                fire(i + 1, 1 - p)

            for c in range(KB):
                if c >= 2:
                    wait_out(c % 2)
                else:

                    @pl.when(i > 0)
                    def _():
                        wait_out(c % 2)

                emit(i, p, c)
            return carry

        lax.fori_loop(0, nb, body, 0)
        wait_out(0)
        wait_out(1)

    return gather_kernel


def kernel(gene_ids, table):
    b, s = gene_ids.shape
    vocab, d = table.shape
    idx_t = gene_ids.T.astype(jnp.int32)
    raw = _make_gather(b, s, vocab, d)(idx_t, table)
    emb = raw.transpose(2, 4, 0, 1, 3).reshape(b, s, d)
    return gene_ids, emb


# speedup vs baseline: 4.8813x; 4.3869x over previous
import functools

import jax
import jax.numpy as jnp
from jax import lax
from jax.experimental import pallas as pl
from jax.experimental.pallas import tpu as pltpu
from jax.experimental.pallas import tpu_sc as plsc

LANES = 128
KB = 4
TPAD = 131


@functools.lru_cache(maxsize=None)
def _make_gather(b: int, s: int, vocab: int, d: int):
    info = plsc.get_sparse_core_info()
    nc, ns = info.num_cores, info.num_subcores
    nw = nc * ns
    assert b % (nw * LANES) == 0 and d % 8 == 0 and s % KB == 0
    nb = s // KB

    @functools.partial(
        pl.kernel,
        mesh=plsc.VectorSubcoreMesh(core_axis_name="c", subcore_axis_name="s"),
        out_type=jax.ShapeDtypeStruct((s, d // 8, b // LANES, 8, LANES), jnp.float32),
        scratch_types=[
            pltpu.VMEM((s, LANES), jnp.int32),
            pltpu.VMEM((2, KB, LANES, d), jnp.float32),
            pltpu.VMEM((2, d // 8, 8, TPAD), jnp.float32),
            pltpu.SemaphoreType.DMA,
            pltpu.SemaphoreType.DMA,
        ],
        compiler_params=pltpu.CompilerParams(
            use_tc_tiling_on_sc=False, needs_layout_passes=False
        ),
    )
    def gather_kernel(idx_hbm, table_hbm, out_hbm, idx_v, gbuf, tbuf, gsem, osem):
        wid = lax.axis_index("s") * nc + lax.axis_index("c")
        pltpu.sync_copy(idx_hbm.at[:, pl.ds(wid * LANES, LANES)], idx_v)

        didx = [jnp.arange(16, dtype=jnp.int32) + 16 * u for u in range(d // 16)]
        tru = [v // 8 for v in didx]
        slu = [v % 8 for v in didx]

        def fire(i, p):
            for c in range(KB):
                pltpu.async_copy(
                    table_hbm.at[idx_v.at[i * KB + c]], gbuf.at[p, c], gsem
                )

        def drain(p):
            for c in range(KB):
                pltpu.make_async_copy(
                    table_hbm.at[idx_v.at[0]], gbuf.at[p, c], gsem
                ).wait()

        def wait_out(q):
            pltpu.make_async_copy(
                out_hbm.at[0, :, wid], tbuf.at[q, :, :, pl.ds(0, LANES)], osem
            ).wait()

        def emit(i, p, c):
            q = c % 2

            @plsc.parallel_loop(0, LANES, 1, unroll=8)
            def tloop(l):
                colv = jnp.full((16,), l, dtype=jnp.int32)
                for u in range(d // 16):
                    vec = gbuf[p, c, l, pl.ds(u * 16, 16)]
                    plsc.store_scatter(tbuf.at[q], [tru[u], slu[u], colv], vec)

            pltpu.async_copy(
                tbuf.at[q, :, :, pl.ds(0, LANES)],
                out_hbm.at[i * KB + c, :, wid],
                osem,
            )

        fire(0, 0)

        def body(i, carry):
            p = i % 2
            drain(p)

            @pl.when(i < nb - 1)
            def _():
                fire(i + 1, 1 - p)

            for c in range(KB):
                if c >= 2:
                    wait_out(c % 2)
                else:

                    @pl.when(i > 0)
                    def _():
                        wait_out(c % 2)

                emit(i, p, c)
            return carry

        lax.fori_loop(0, nb, body, 0)
        wait_out(0)
        wait_out(1)

    return gather_kernel


def kernel(gene_ids, table):
    b, s = gene_ids.shape
    vocab, d = table.shape
    idx_t = gene_ids.T.astype(jnp.int32)
    raw = _make_gather(b, s, vocab, d)(idx_t, table)
    emb = raw.transpose(2, 4, 0, 1, 3).reshape(b, s, d)
    return gene_ids, emb


# speedup vs baseline: 4.9228x; 1.0085x over previous
import functools

import jax
import jax.numpy as jnp
from jax import lax
from jax.experimental import pallas as pl
from jax.experimental.pallas import tpu as pltpu
from jax.experimental.pallas import tpu_sc as plsc

LANES = 128
KB = 4
TPAD = 131


@functools.lru_cache(maxsize=None)
def _make_gather(b: int, s: int, vocab: int, d: int):
    info = plsc.get_sparse_core_info()
    nc, ns = info.num_cores, info.num_subcores
    nw = nc * ns
    assert b % (nw * LANES) == 0 and d % 8 == 0 and s % KB == 0
    nb = s // KB

    @functools.partial(
        pl.kernel,
        mesh=plsc.VectorSubcoreMesh(core_axis_name="c", subcore_axis_name="s"),
        out_type=jax.ShapeDtypeStruct((s, d // 8, b // LANES, 8, LANES), jnp.float32),
        scratch_types=[
            pltpu.VMEM((s, LANES), jnp.int32),
            pltpu.VMEM((2, KB, LANES, d), jnp.float32),
            pltpu.VMEM((2, d // 8, 8, TPAD), jnp.float32),
            pltpu.SemaphoreType.DMA((2,)),
            pltpu.SemaphoreType.DMA,
        ],
        compiler_params=pltpu.CompilerParams(
            use_tc_tiling_on_sc=False, needs_layout_passes=False
        ),
    )
    def gather_kernel(idx_hbm, table_hbm, out_hbm, idx_v, gbuf, tbuf, gsem, osem):
        wid = lax.axis_index("s") * nc + lax.axis_index("c")
        pltpu.sync_copy(idx_hbm.at[:, pl.ds(wid * LANES, LANES)], idx_v)

        didx = [jnp.arange(16, dtype=jnp.int32) + 16 * u for u in range(d // 16)]
        tru = [v // 8 for v in didx]
        slu = [v % 8 for v in didx]

        def fire(i, p):
            for c in range(KB):
                pltpu.async_copy(
                    table_hbm.at[idx_v.at[i * KB + c]], gbuf.at[p, c], gsem.at[p]
                )

        def drain(p):
            for c in range(KB):
                pltpu.make_async_copy(
                    table_hbm.at[idx_v.at[0]], gbuf.at[p, c], gsem.at[p]
                ).wait()

        def wait_out(q):
            pltpu.make_async_copy(
                out_hbm.at[0, :, wid], tbuf.at[q, :, :, pl.ds(0, LANES)], osem
            ).wait()

        def emit(i, p, c):
            q = c % 2

            @plsc.parallel_loop(0, LANES, 1, unroll=8)
            def tloop(l):
                colv = jnp.full((16,), l, dtype=jnp.int32)
                for u in range(d // 16):
                    vec = gbuf[p, c, l, pl.ds(u * 16, 16)]
                    plsc.store_scatter(tbuf.at[q], [tru[u], slu[u], colv], vec)

            pltpu.async_copy(
                tbuf.at[q, :, :, pl.ds(0, LANES)],
                out_hbm.at[i * KB + c, :, wid],
                osem,
            )

        fire(0, 0)

        def body(i, carry):
            p = i % 2

            @pl.when(i < nb - 1)
            def _():
                fire(i + 1, 1 - p)

            drain(p)

            for c in range(KB):
                if c >= 2:
                    wait_out(c % 2)
                else:

                    @pl.when(i > 0)
                    def _():
                        wait_out(c % 2)

                emit(i, p, c)
            return carry

        lax.fori_loop(0, nb, body, 0)
        wait_out(0)
        wait_out(1)

    return gather_kernel


def kernel(gene_ids, table):
    b, s = gene_ids.shape
    vocab, d = table.shape
    idx_t = gene_ids.T.astype(jnp.int32)
    raw = _make_gather(b, s, vocab, d)(idx_t, table)
    emb = raw.transpose(2, 4, 0, 1, 3).reshape(b, s, d)
    return gene_ids, emb
